# Initial kernel scaffold; baseline (speedup 1.0000x reference)
import jax
import jax.numpy as jnp
from jax.experimental import pallas as pl


def kernel(vi, atom_mol_batch, N, W, b):
    raise NotImplementedError("write your pallas kernel here")


# speedup vs baseline: 2.3175x; 2.3175x over previous
import jax
import jax.numpy as jnp
from jax import lax
from jax.experimental import pallas as pl
from jax.experimental.pallas import tpu as pltpu
from jax.experimental.pallas import tpu_sc as plsc

N_ATOMS_C = 320000
N_MOLS_C = 10000
D = 128

NC = 2
NS = 16
NW = NC * NS

MH = N_MOLS_C // NC
TRASH = 5120
MLOC = 5248
STRIPE = TRASH // NS

CH = 80
NCHT = N_ATOMS_C // CH

MMB = 1280


def _mm_body(x_ref, w_ref, b_ref, o_ref):
    y = jnp.dot(x_ref[...], w_ref[...], preferred_element_type=jnp.float32)
    o_ref[...] = jnp.maximum(y + b_ref[...], 0.0)


def _matmul_relu(x, w, b2):
    n = x.shape[0]
    return pl.pallas_call(
        _mm_body,
        grid=(n // MMB,),
        in_specs=[
            pl.BlockSpec((MMB, D), lambda i: (i, 0)),
            pl.BlockSpec((D, D), lambda i: (0, 0)),
            pl.BlockSpec((1, D), lambda i: (0, 0)),
        ],
        out_specs=pl.BlockSpec((MMB, D), lambda i: (i, 0)),
        out_shape=jax.ShapeDtypeStruct((n, D), jnp.float32),
    )(x, w, b2)


def _sc_body(prop_h, idx_h, meta_h, zrows_h, zhist_h,
             part_h, hists_h,
             acc_sh, idx_v, rows_v, hist_v, meta_v):
    c = lax.axis_index("c")
    s = lax.axis_index("s")
    wid = c * NS + s

    pltpu.sync_copy(meta_h.at[wid], meta_v)
    mv = meta_v[...]
    iota16 = lax.iota(jnp.int32, 16)
    t0 = jnp.sum(jnp.where(iota16 == 0, mv, 0))
    cnt = jnp.sum(jnp.where(iota16 == 1, mv, 0))
    lo_m = c * MH

    pltpu.sync_copy(zhist_h, hist_v)
    pltpu.sync_copy(zrows_h, rows_v)
    z0 = s * STRIPE
    for k in range(STRIPE // CH):
        pltpu.sync_copy(rows_v, acc_sh.at[pl.ds(z0 + k * CH, CH), :])
    pltpu.sync_copy(rows_v, acc_sh.at[pl.ds(TRASH, CH), :])
    plsc.subcore_barrier()

    ones16 = jnp.ones((16,), jnp.float32)

    def body(j, carry):
        base = (t0 + j) * CH
        pltpu.sync_copy(idx_h.at[pl.ds(base, CH)], idx_v)
        pltpu.sync_copy(prop_h.at[pl.ds(base, CH), :], rows_v)
        for g in range(CH // 16):
            iv = idx_v[pl.ds(g * 16, 16)] - lo_m
            ok = (iv >= 0) & (iv < MH)
            ivc = jnp.where(ok, iv, TRASH)
            idx_v[pl.ds(g * 16, 16)] = ivc
            plsc.addupdate_scatter(hist_v, [ivc], ones16)
        pltpu.sync_copy(rows_v, acc_sh.at[idx_v], add=True)
        return carry

    lax.fori_loop(0, cnt, body, 0)
    plsc.subcore_barrier()

    for k in range(STRIPE // CH):
        pltpu.sync_copy(acc_sh.at[pl.ds(z0 + k * CH, CH), :], rows_v)
        pltpu.sync_copy(rows_v, part_h.at[c, pl.ds(z0 + k * CH, CH), :])
    pltpu.sync_copy(hist_v, hists_h.at[wid])


def _sc_segment_sum(prop, idx, meta):
    mesh = plsc.VectorSubcoreMesh(core_axis_name="c", subcore_axis_name="s")
    f = pl.kernel(
        _sc_body,
        out_type=(
            jax.ShapeDtypeStruct((NC, TRASH, D), jnp.float32),
            jax.ShapeDtypeStruct((NW, MLOC), jnp.float32),
        ),
        mesh=mesh,
        compiler_params=pltpu.CompilerParams(needs_layout_passes=False),
        scratch_types=[
            pltpu.VMEM_SHARED((MLOC, D), jnp.float32),
            pltpu.VMEM((CH,), jnp.int32),
            pltpu.VMEM((CH, D), jnp.float32),
            pltpu.VMEM((MLOC,), jnp.float32),
            pltpu.VMEM((16,), jnp.int32),
        ],
    )
    zrows = jnp.zeros((CH, D), jnp.float32)
    zhist = jnp.zeros((MLOC,), jnp.float32)
    return f(prop, idx, meta, zrows, zhist)


def _merge_body(p_ref, h_ref, o_ref):
    cnt = jnp.sum(h_ref[0], axis=1, keepdims=True)
    o_ref[...] = p_ref[0] / jnp.maximum(cnt, 1.0)


def _merge(part, hists3, n_mols):
    mb = 1000
    nb = MH // mb
    return pl.pallas_call(
        _merge_body,
        grid=(n_mols // mb,),
        in_specs=[
            pl.BlockSpec((1, mb, D), lambda i: (i // nb, i % nb, 0)),
            pl.BlockSpec((1, mb, NS), lambda i: (i // nb, i % nb, 0)),
        ],
        out_specs=pl.BlockSpec((mb, D), lambda i: (i, 0)),
        out_shape=jax.ShapeDtypeStruct((n_mols, D), jnp.float32),
    )(part, hists3)


def kernel(vi, atom_mol_batch, N, W, b):
    n_mols = N.shape[0]
    n_atoms = atom_mol_batch.shape[0]
    assert n_atoms == N_ATOMS_C and n_mols == N_MOLS_C and vi.shape[1] == D

    split = jnp.searchsorted(atom_mol_batch, MH).astype(jnp.int32)
    ksplit = split // CH
    lo = jnp.stack([jnp.int32(0), jnp.minimum(ksplit, NCHT)])
    hi = jnp.stack([jnp.minimum(ksplit + 1, NCHT), jnp.int32(NCHT)])
    ln = hi - lo
    s_ids = jnp.arange(NS, dtype=jnp.int32)
    t0 = lo[:, None] + (s_ids[None, :] * ln[:, None]) // NS
    t1 = lo[:, None] + ((s_ids[None, :] + 1) * ln[:, None]) // NS
    meta = jnp.zeros((NW, 16), jnp.int32)
    meta = meta.at[:, 0].set(t0.reshape(-1))
    meta = meta.at[:, 1].set((t1 - t0).reshape(-1))

    atom_prop = _matmul_relu(vi[:-n_mols], W, b.reshape(1, D))
    part, hists = _sc_segment_sum(atom_prop, atom_mol_batch, meta)
    hists3 = hists.reshape(NC, NS, MLOC).transpose(0, 2, 1)
    return _merge(part, hists3, n_mols)


# speedup vs baseline: 2.5032x; 1.0801x over previous
import jax
import jax.numpy as jnp
from jax import lax
from jax.experimental import pallas as pl
from jax.experimental.pallas import tpu as pltpu
from jax.experimental.pallas import tpu_sc as plsc

N_ATOMS_C = 320000
N_MOLS_C = 10000
D = 128

NC = 2
NS = 16
NW = NC * NS

MH = N_MOLS_C // NC
TRASH = 5120
MLOC = 5248
STRIPE = TRASH // NS

CH = 128
NCHT = N_ATOMS_C // CH

MMB = 1280


def _mm_body(x_ref, w_ref, b_ref, o_ref):
    y = jnp.dot(x_ref[...], w_ref[...], preferred_element_type=jnp.float32)
    o_ref[...] = jnp.maximum(y + b_ref[...], 0.0)


def _matmul_relu(x, w, b2):
    n = x.shape[0]
    return pl.pallas_call(
        _mm_body,
        grid=(n // MMB,),
        in_specs=[
            pl.BlockSpec((MMB, D), lambda i: (i, 0)),
            pl.BlockSpec((D, D), lambda i: (0, 0)),
            pl.BlockSpec((1, D), lambda i: (0, 0)),
        ],
        out_specs=pl.BlockSpec((MMB, D), lambda i: (i, 0)),
        out_shape=jax.ShapeDtypeStruct((n, D), jnp.float32),
    )(x, w, b2)


def _sc_body(prop_h, idx_h, meta_h, zrows_h, zhist_h,
             part_h, hists_h,
             acc_sh, idx_v, rows_v, hist_v, meta_v):
    c = lax.axis_index("c")
    s = lax.axis_index("s")
    wid = c * NS + s

    pltpu.sync_copy(meta_h.at[wid], meta_v)
    mv = meta_v[...]
    iota16 = lax.iota(jnp.int32, 16)
    t0 = jnp.sum(jnp.where(iota16 == 0, mv, 0))
    cnt = jnp.sum(jnp.where(iota16 == 1, mv, 0))
    lo_m = c * MH

    pltpu.sync_copy(zhist_h, hist_v)
    pltpu.sync_copy(zrows_h, rows_v)
    z0 = s * STRIPE
    for k in range(STRIPE // 80):
        pltpu.sync_copy(rows_v.at[pl.ds(0, 80), :], acc_sh.at[pl.ds(z0 + k * 80, 80), :])
    pltpu.sync_copy(rows_v, acc_sh.at[pl.ds(TRASH, CH), :])
    plsc.subcore_barrier()

    ones16 = jnp.ones((16,), jnp.float32)

    def body(j, carry):
        base = (t0 + j) * CH
        pltpu.sync_copy(idx_h.at[pl.ds(base, CH)], idx_v)
        pltpu.sync_copy(prop_h.at[pl.ds(base, CH), :], rows_v)
        for g in range(CH // 16):
            iv = idx_v[pl.ds(g * 16, 16)] - lo_m
            ok = (iv >= 0) & (iv < MH)
            ivc = jnp.where(ok, iv, TRASH)
            idx_v[pl.ds(g * 16, 16)] = ivc
            plsc.addupdate_scatter(hist_v, [ivc], ones16)
        pltpu.sync_copy(rows_v, acc_sh.at[idx_v], add=True)
        return carry

    lax.fori_loop(0, cnt, body, 0)
    plsc.subcore_barrier()

    for k in range(STRIPE // 80):
        pltpu.sync_copy(acc_sh.at[pl.ds(z0 + k * 80, 80), :], rows_v.at[pl.ds(0, 80), :])
        pltpu.sync_copy(rows_v.at[pl.ds(0, 80), :], part_h.at[c, pl.ds(z0 + k * 80, 80), :])
    pltpu.sync_copy(hist_v, hists_h.at[wid])


def _sc_segment_sum(prop, idx, meta):
    mesh = plsc.VectorSubcoreMesh(core_axis_name="c", subcore_axis_name="s")
    f = pl.kernel(
        _sc_body,
        out_type=(
            jax.ShapeDtypeStruct((NC, TRASH, D), jnp.float32),
            jax.ShapeDtypeStruct((NW, MLOC), jnp.float32),
        ),
        mesh=mesh,
        compiler_params=pltpu.CompilerParams(needs_layout_passes=False),
        scratch_types=[
            pltpu.VMEM_SHARED((MLOC, D), jnp.float32),
            pltpu.VMEM((CH,), jnp.int32),
            pltpu.VMEM((CH, D), jnp.float32),
            pltpu.VMEM((MLOC,), jnp.float32),
            pltpu.VMEM((16,), jnp.int32),
        ],
    )
    zrows = jnp.zeros((CH, D), jnp.float32)
    zhist = jnp.zeros((MLOC,), jnp.float32)
    return f(prop, idx, meta, zrows, zhist)


def _merge_body(p_ref, h_ref, o_ref):
    cnt = jnp.sum(h_ref[0], axis=1, keepdims=True)
    o_ref[...] = p_ref[0] / jnp.maximum(cnt, 1.0)


def _merge(part, hists3, n_mols):
    mb = 1000
    nb = MH // mb
    return pl.pallas_call(
        _merge_body,
        grid=(n_mols // mb,),
        in_specs=[
            pl.BlockSpec((1, mb, D), lambda i: (i // nb, i % nb, 0)),
            pl.BlockSpec((1, mb, NS), lambda i: (i // nb, i % nb, 0)),
        ],
        out_specs=pl.BlockSpec((mb, D), lambda i: (i, 0)),
        out_shape=jax.ShapeDtypeStruct((n_mols, D), jnp.float32),
    )(part, hists3)


def kernel(vi, atom_mol_batch, N, W, b):
    n_mols = N.shape[0]
    n_atoms = atom_mol_batch.shape[0]
    assert n_atoms == N_ATOMS_C and n_mols == N_MOLS_C and vi.shape[1] == D

    split = jnp.searchsorted(atom_mol_batch, MH).astype(jnp.int32)
    ksplit = split // CH
    lo = jnp.stack([jnp.int32(0), jnp.minimum(ksplit, NCHT)])
    hi = jnp.stack([jnp.minimum(ksplit + 1, NCHT), jnp.int32(NCHT)])
    ln = hi - lo
    s_ids = jnp.arange(NS, dtype=jnp.int32)
    t0 = lo[:, None] + (s_ids[None, :] * ln[:, None]) // NS
    t1 = lo[:, None] + ((s_ids[None, :] + 1) * ln[:, None]) // NS
    meta = jnp.zeros((NW, 16), jnp.int32)
    meta = meta.at[:, 0].set(t0.reshape(-1))
    meta = meta.at[:, 1].set((t1 - t0).reshape(-1))

    atom_prop = _matmul_relu(vi[:-n_mols], W, b.reshape(1, D))
    part, hists = _sc_segment_sum(atom_prop, atom_mol_batch, meta)
    hists3 = hists.reshape(NC, NS, MLOC).transpose(0, 2, 1)
    return _merge(part, hists3, n_mols)
